# baseline (speedup 1.0000x reference)
import functools

import jax
import jax.numpy as jnp
from jax import lax
from jax.experimental import pallas as pl
from jax.experimental.pallas import tpu as pltpu
from jax.experimental.pallas import tpu_sc as plsc

CHUNK = 128
WSPAN = 2
NWBUF = 3


@functools.lru_cache(maxsize=None)
def _emb_lookup(B, V, D):
    info = plsc.get_sparse_core_info()
    NC, NS = info.num_cores, info.num_subcores
    NW = NC * NS
    assert B % (NW * CHUNK) == 0
    b_per_w = B // NW
    nchunks = b_per_w // CHUNK
    nspans = nchunks // WSPAN
    rows_per_span = WSPAN * CHUNK
    mesh = plsc.VectorSubcoreMesh(core_axis_name="c", subcore_axis_name="s")

    @functools.partial(
        pl.kernel,
        mesh=mesh,
        out_type=jax.ShapeDtypeStruct((B, D), jnp.float32),
        scratch_types=[
            pltpu.VMEM((nchunks, CHUNK), jnp.int32),
            pltpu.VMEM((NWBUF, rows_per_span, D), jnp.float32),
            pltpu.VMEM_SHARED((V, D), jnp.float32),
            [pltpu.SemaphoreType.DMA] * NWBUF,
            [pltpu.SemaphoreType.DMA] * NWBUF,
        ],
    )
    def k(x_hbm, table_hbm, out_hbm, idx_v, rows_v, table_sp, gsems, osems):
        sid = lax.axis_index("s")
        wid = sid * NC + lax.axis_index("c")
        base = wid * b_per_w

        @pl.when(sid == 0)
        def _():
            pltpu.sync_copy(table_hbm, table_sp)

        pltpu.sync_copy(x_hbm.at[wid], idx_v)
        plsc.subcore_barrier()

        def gathers(s, w):
            for half in range(WSPAN):
                pltpu.async_copy(
                    table_sp.at[idx_v.at[s * WSPAN + half]],
                    rows_v.at[w].at[pl.ds(half * CHUNK, CHUNK)],
                    gsems[w],
                )

        def wait_gathers(s, w):
            for half in range(WSPAN):
                pltpu.make_async_copy(
                    table_sp.at[idx_v.at[s * WSPAN + half]],
                    rows_v.at[w].at[pl.ds(half * CHUNK, CHUNK)],
                    gsems[w],
                ).wait()

        def write(s, w):
            pltpu.async_copy(
                rows_v.at[w],
                out_hbm.at[pl.ds(base + s * rows_per_span, rows_per_span)],
                osems[w],
            )

        def wait_write(s, w):
            pltpu.make_async_copy(
                rows_v.at[w],
                out_hbm.at[pl.ds(base + s * rows_per_span, rows_per_span)],
                osems[w],
            ).wait()

        nfull = (nspans // NWBUF) * NWBUF
        for w in range(NWBUF):
            gathers(w, w)

        def body(i, carry):
            s0 = i * NWBUF
            for w in range(NWBUF):
                wait_gathers(s0 + w, w)
                write(s0 + w, w)
            for w in range(NWBUF):
                nxt = s0 + NWBUF + w

                @pl.when(nxt < nspans)
                def _():
                    wait_write(s0 + w, w)
                    gathers(nxt, w)

            return carry

        lax.fori_loop(0, nfull // NWBUF, body, 0)

        for s in range(nfull, nspans):
            w = s % NWBUF
            wait_gathers(s, w)
            write(s, w)
        for w in range(NWBUF):
            s_last = nspans - NWBUF + w
            wait_write(s_last, w)

    return k


def kernel(x, table):
    NB, H = x.shape
    V, D = table.shape
    B = NB * H
    info = plsc.get_sparse_core_info()
    NW = info.num_cores * info.num_subcores
    xr = x.T.reshape(NW, B // (NW * CHUNK), CHUNK)
    out = _emb_lookup(B, V, D)(xr, table)
    return out.reshape(H, NB, D).transpose(1, 0, 2)


# speedup vs baseline: 1.0673x; 1.0673x over previous
import functools

import jax
import jax.numpy as jnp
from jax import lax
from jax.experimental import pallas as pl
from jax.experimental.pallas import tpu as pltpu
from jax.experimental.pallas import tpu_sc as plsc

CHUNK = 64
NBUF = 10


@functools.lru_cache(maxsize=None)
def _emb_lookup(B, V, D):
    info = plsc.get_sparse_core_info()
    NC, NS = info.num_cores, info.num_subcores
    NW = NC * NS
    assert B % (NW * CHUNK) == 0
    b_per_w = B // NW
    nchunks = b_per_w // CHUNK
    assert nchunks % NBUF == 0
    nrounds = nchunks // NBUF
    mesh = plsc.VectorSubcoreMesh(core_axis_name="c", subcore_axis_name="s")

    @functools.partial(
        pl.kernel,
        mesh=mesh,
        out_type=jax.ShapeDtypeStruct((B, D), jnp.float32),
        scratch_types=[
            pltpu.VMEM((nchunks, CHUNK), jnp.int32),
            pltpu.VMEM((NBUF, CHUNK, D), jnp.float32),
            pltpu.VMEM_SHARED((V, D), jnp.float32),
            [pltpu.SemaphoreType.DMA] * NBUF,
            [pltpu.SemaphoreType.DMA] * NBUF,
        ],
    )
    def k(x_hbm, table_hbm, out_hbm, idx_v, rows_v, table_sp, gsems, osems):
        sid = lax.axis_index("s")
        wid = sid * NC + lax.axis_index("c")
        base = wid * b_per_w

        @pl.when(sid == 0)
        def _():
            pltpu.sync_copy(table_hbm, table_sp)

        pltpu.sync_copy(x_hbm.at[wid], idx_v)
        plsc.subcore_barrier()

        def gather(g, b):
            return pltpu.async_copy(
                table_sp.at[idx_v.at[g]], rows_v.at[b], gsems[b]
            )

        def write(g, b):
            return pltpu.async_copy(
                rows_v.at[b], out_hbm.at[pl.ds(base + g * CHUNK, CHUNK)], osems[b]
            )

        for b in range(NBUF):
            gather(b, b)

        def body(i, carry):
            g0 = i * NBUF
            for b in range(NBUF):
                pltpu.make_async_copy(
                    table_sp.at[idx_v.at[g0 + b]], rows_v.at[b], gsems[b]
                ).wait()
                write(g0 + b, b)

            @pl.when(i + 1 < nrounds)
            def _():
                for b in range(NBUF):
                    pltpu.make_async_copy(
                        rows_v.at[b],
                        out_hbm.at[pl.ds(base + (g0 + b) * CHUNK, CHUNK)],
                        osems[b],
                    ).wait()
                    gather(g0 + NBUF + b, b)

            return carry

        lax.fori_loop(0, nrounds, body, 0)

        for b in range(NBUF):
            g = nchunks - NBUF + b
            pltpu.make_async_copy(
                rows_v.at[b],
                out_hbm.at[pl.ds(base + g * CHUNK, CHUNK)],
                osems[b],
            ).wait()

    return k


def kernel(x, table):
    NB, H = x.shape
    V, D = table.shape
    B = NB * H
    info = plsc.get_sparse_core_info()
    NW = info.num_cores * info.num_subcores
    xr = x.T.reshape(NW, B // (NW * CHUNK), CHUNK)
    out = _emb_lookup(B, V, D)(xr, table)
    return out.reshape(H, NB, D).transpose(1, 0, 2)
